# baseline (speedup 1.0000x reference)
import functools

import jax
import jax.numpy as jnp
from jax import lax
from jax.experimental import pallas as pl
from jax.experimental.pallas import tpu as pltpu
from jax.experimental.pallas import tpu_sc as plsc

N_NODES = 10000
N_EDGES = 320000
CH = 128
HCH = CH // 2
CNTW = 16
NC = 2
NS = 16
E_PER_S = N_EDGES // NS
WIN = 80
NWIN = E_PER_S // WIN
CWIN = 25
NCHUNK = NWIN // CWIN
ROWS_PER_SUB = 632
N_PAD = NS * ROWS_PER_SUB
LAST_ROWS = N_NODES - (NS - 1) * ROWS_PER_SUB


def _sc_aggregate(x, src3, dst3, zf, zc):
    @functools.partial(
        pl.kernel,
        out_type=(
            jax.ShapeDtypeStruct((N_PAD, CH), jnp.float32),
            jax.ShapeDtypeStruct((NC, N_PAD, CNTW), jnp.float32),
        ),
        mesh=plsc.VectorSubcoreMesh(core_axis_name="c", subcore_axis_name="s"),
        compiler_params=pltpu.CompilerParams(use_tc_tiling_on_sc=False),
        scratch_types=[
            pltpu.VMEM((CWIN, WIN), jnp.int32),
            pltpu.VMEM((CWIN, WIN), jnp.int32),
            pltpu.VMEM((WIN, HCH), jnp.float32),
            pltpu.VMEM((WIN, HCH), jnp.float32),
            pltpu.VMEM((WIN, HCH), jnp.float32),
            pltpu.VMEM((WIN, CNTW), jnp.float32),
            pltpu.VMEM_SHARED((N_PAD, HCH), jnp.float32),
            pltpu.VMEM_SHARED((N_PAD, HCH), jnp.float32),
            pltpu.VMEM_SHARED((N_PAD, CNTW), jnp.float32),
            pltpu.SemaphoreType.DMA,
            pltpu.SemaphoreType.DMA,
            pltpu.SemaphoreType.DMA,
            pltpu.SemaphoreType.DMA,
            pltpu.SemaphoreType.DMA,
            pltpu.SemaphoreType.DMA,
            pltpu.SemaphoreType.DMA,
            pltpu.SemaphoreType.DMA,
            pltpu.SemaphoreType.DMA,
        ],
    )
    def agg(x_hbm, src_hbm, dst_hbm, zf_hbm, zc_hbm, out_hbm, cnt_hbm,
            src_v, dst_v, rows_v0, rows_v1, rows_v2, ones_v, x_sh, acc_sh,
            cnt_sh, g0, g1, g2, s0, s1, s2, c0, c1, c2):
        cid = lax.axis_index("c")
        sid = lax.axis_index("s")
        stripe = pl.multiple_of(sid * ROWS_PER_SUB, 8)
        col0 = pl.multiple_of(cid * HCH, 8)
        rows = (rows_v0, rows_v1, rows_v2)
        gsem = (g0, g1, g2)
        ssem = (s0, s1, s2)
        csem = (c0, c1, c2)

        zf_cp = pltpu.async_copy(
            zf_hbm, acc_sh.at[pl.ds(stripe, ROWS_PER_SUB)], g0)
        zc_cp = pltpu.async_copy(
            zc_hbm, cnt_sh.at[pl.ds(stripe, ROWS_PER_SUB)], g1)

        @pl.when(sid < NS - 1)
        def _():
            pltpu.sync_copy(
                x_hbm.at[pl.ds(stripe, ROWS_PER_SUB), pl.ds(col0, HCH)],
                x_sh.at[pl.ds(stripe, ROWS_PER_SUB)])

        @pl.when(sid == NS - 1)
        def _():
            pltpu.sync_copy(
                x_hbm.at[pl.ds(stripe, LAST_ROWS), pl.ds(col0, HCH)],
                x_sh.at[pl.ds(stripe, LAST_ROWS)])

        @pl.loop(0, WIN)
        def _(i):
            ones_v[i, :] = jnp.ones((CNTW,), jnp.float32)

        zf_cp.wait()
        zc_cp.wait()
        plsc.subcore_barrier()

        def fire_gather(w, b):
            pltpu.async_copy(x_sh.at[src_v.at[w]], rows[b], gsem[b])

        def wait_gather(w, b):
            pltpu.make_async_copy(
                x_sh.at[src_v.at[w]], rows[b], gsem[b]).wait()

        def start_scatter(w, b, cnt_here):
            pltpu.async_copy(rows[b], acc_sh.at[dst_v.at[w]], ssem[b],
                             add=True)

            @pl.when(cnt_here)
            def _():
                pltpu.async_copy(ones_v, cnt_sh.at[dst_v.at[w]], csem[b],
                                 add=True)

        def wait_scatter(w, b, cnt_here):
            pltpu.make_async_copy(rows[b], acc_sh.at[dst_v.at[w]],
                                  ssem[b]).wait()

            @pl.when(cnt_here)
            def _():
                pltpu.make_async_copy(ones_v, cnt_sh.at[dst_v.at[w]],
                                      csem[b]).wait()

        for k in range(NCHUNK):
            cnt_here = cid == (0 if k < NCHUNK // 2 else 1)
            pltpu.sync_copy(src_hbm.at[sid * NCHUNK + k], src_v)
            pltpu.sync_copy(dst_hbm.at[sid * NCHUNK + k], dst_v)

            fire_gather(0, 0)
            fire_gather(1, 1)
            wait_gather(0, 0)
            start_scatter(0, 0, cnt_here)
            fire_gather(2, 2)
            wait_gather(1, 1)
            start_scatter(1, 1, cnt_here)
            wait_scatter(0, 0, cnt_here)
            fire_gather(3, 0)

            @pl.loop(0, (CWIN - 4) // 3)
            def _(q):
                for db in range(3):
                    w = 2 + q * 3 + db
                    b = (2 + db) % 3
                    wait_gather(w, b)
                    start_scatter(w, b, cnt_here)
                    wait_scatter(w - 1, (b + 2) % 3, cnt_here)
                    fire_gather(w + 2, (b + 2) % 3)

            wait_gather(CWIN - 2, (CWIN - 2) % 3)
            start_scatter(CWIN - 2, (CWIN - 2) % 3, cnt_here)
            wait_scatter(CWIN - 3, (CWIN - 3) % 3, cnt_here)
            wait_gather(CWIN - 1, (CWIN - 1) % 3)
            start_scatter(CWIN - 1, (CWIN - 1) % 3, cnt_here)
            wait_scatter(CWIN - 2, (CWIN - 2) % 3, cnt_here)
            wait_scatter(CWIN - 1, (CWIN - 1) % 3, cnt_here)

        plsc.subcore_barrier()
        pltpu.sync_copy(
            acc_sh.at[pl.ds(stripe, ROWS_PER_SUB)],
            out_hbm.at[pl.ds(stripe, ROWS_PER_SUB), pl.ds(col0, HCH)])
        pltpu.sync_copy(
            cnt_sh.at[pl.ds(stripe, ROWS_PER_SUB)],
            cnt_hbm.at[cid, pl.ds(stripe, ROWS_PER_SUB)])

    return agg(x, src3, dst3, zf, zc)


def _tc_finish(acc, cnt, x, W_l, b_l, W_r):
    R = 1000

    def body(acc_ref, cnt_ref, x_ref, wl_ref, bl_ref, wr_ref, o_ref):
        counts = (cnt_ref[0] + cnt_ref[1])[:, :1]
        mean = acc_ref[...] / jnp.maximum(counts, 1.0)
        z = jnp.dot(mean, wl_ref[...], preferred_element_type=jnp.float32)
        z = z + bl_ref[...] + jnp.dot(x_ref[...], wr_ref[...],
                                      preferred_element_type=jnp.float32)
        o_ref[...] = jnp.maximum(z, 0.0)

    return pl.pallas_call(
        body,
        grid=(N_NODES // R,),
        in_specs=[
            pl.BlockSpec((R, CH), lambda i: (i, 0)),
            pl.BlockSpec((NC, R, CNTW), lambda i: (0, i, 0)),
            pl.BlockSpec((R, CH), lambda i: (i, 0)),
            pl.BlockSpec((CH, CH), lambda i: (0, 0)),
            pl.BlockSpec((1, CH), lambda i: (0, 0)),
            pl.BlockSpec((CH, CH), lambda i: (0, 0)),
        ],
        out_specs=pl.BlockSpec((R, CH), lambda i: (i, 0)),
        out_shape=jax.ShapeDtypeStruct((N_NODES, CH), jnp.float32),
    )(acc, cnt, x, W_l, b_l.reshape(1, CH), W_r)


def kernel(x, edge_index, W_l, b_l, W_r):
    src3 = edge_index[0].reshape(NS * NCHUNK, CWIN, WIN)
    dst3 = edge_index[1].reshape(NS * NCHUNK, CWIN, WIN)
    zf = jnp.zeros((ROWS_PER_SUB, HCH), jnp.float32)
    zc = jnp.zeros((ROWS_PER_SUB, CNTW), jnp.float32)
    acc, cnt = _sc_aggregate(x, src3, dst3, zf, zc)
    return _tc_finish(acc, cnt, x, W_l, b_l, W_r)


# speedup vs baseline: 1.3189x; 1.3189x over previous
import functools

import jax
import jax.numpy as jnp
from jax import lax
from jax.experimental import pallas as pl
from jax.experimental.pallas import tpu as pltpu
from jax.experimental.pallas import tpu_sc as plsc

N_NODES = 10000
N_EDGES = 320000
CH = 128
CNTW = 16
NC = 2
NS = 16
NW = NC * NS
E_PER_W = N_EDGES // NW
WIN = 80
NWIN = E_PER_W // WIN
CWIN = 25
NCHUNK = NWIN // CWIN
ROWS_PER_SUB = 632
N_PAD = NS * ROWS_PER_SUB


def _sc_aggregate(x, src3, dst3, zf, zc):
    @functools.partial(
        pl.kernel,
        out_type=(
            jax.ShapeDtypeStruct((NC, N_PAD, CH), jnp.float32),
            jax.ShapeDtypeStruct((NC, N_PAD, CNTW), jnp.float32),
        ),
        mesh=plsc.VectorSubcoreMesh(core_axis_name="c", subcore_axis_name="s"),
        compiler_params=pltpu.CompilerParams(use_tc_tiling_on_sc=False),
        scratch_types=[
            pltpu.VMEM((CWIN, WIN), jnp.int32),
            pltpu.VMEM((CWIN, WIN), jnp.int32),
            pltpu.VMEM((WIN, CH), jnp.float32),
            pltpu.VMEM((WIN, CH), jnp.float32),
            pltpu.VMEM((WIN, CH), jnp.float32),
            pltpu.VMEM((WIN, CNTW), jnp.float32),
            pltpu.VMEM_SHARED((N_PAD, CH), jnp.float32),
            pltpu.VMEM_SHARED((N_PAD, CNTW), jnp.float32),
            pltpu.SemaphoreType.DMA,
            pltpu.SemaphoreType.DMA,
            pltpu.SemaphoreType.DMA,
            pltpu.SemaphoreType.DMA,
            pltpu.SemaphoreType.DMA,
            pltpu.SemaphoreType.DMA,
            pltpu.SemaphoreType.DMA,
            pltpu.SemaphoreType.DMA,
            pltpu.SemaphoreType.DMA,
        ],
    )
    def agg(x_hbm, src_hbm, dst_hbm, zf_hbm, zc_hbm, out_hbm, cnt_hbm,
            src_v, dst_v, rows_v0, rows_v1, rows_v2, ones_v, acc_sh, cnt_sh,
            g0, g1, g2, s0, s1, s2, c0, c1, c2):
        cid = lax.axis_index("c")
        sid = lax.axis_index("s")
        wid = cid * NS + sid
        stripe = pl.multiple_of(sid * ROWS_PER_SUB, 8)
        rows = (rows_v0, rows_v1, rows_v2)
        gsem = (g0, g1, g2)
        ssem = (s0, s1, s2)
        csem = (c0, c1, c2)

        zf_cp = pltpu.async_copy(
            zf_hbm, acc_sh.at[pl.ds(stripe, ROWS_PER_SUB)], g0)
        zc_cp = pltpu.async_copy(
            zc_hbm, cnt_sh.at[pl.ds(stripe, ROWS_PER_SUB)], g1)
        pltpu.sync_copy(src_hbm.at[wid], src_v)
        pltpu.sync_copy(dst_hbm.at[wid], dst_v)

        @pl.loop(0, WIN)
        def _(i):
            ones_v[i, :] = jnp.ones((CNTW,), jnp.float32)

        zf_cp.wait()
        zc_cp.wait()
        plsc.subcore_barrier()

        def fire_gather(w, b):
            pltpu.async_copy(x_hbm.at[src_v.at[w]], rows[b], gsem[b])

        def wait_gather(w, b):
            pltpu.make_async_copy(
                x_hbm.at[src_v.at[w]], rows[b], gsem[b]).wait()

        def start_scatter(w, b):
            pltpu.async_copy(rows[b], acc_sh.at[dst_v.at[w]], ssem[b],
                             add=True)
            pltpu.async_copy(ones_v, cnt_sh.at[dst_v.at[w]], csem[b],
                             add=True)

        def wait_scatter(w, b):
            pltpu.make_async_copy(rows[b], acc_sh.at[dst_v.at[w]],
                                  ssem[b]).wait()
            pltpu.make_async_copy(ones_v, cnt_sh.at[dst_v.at[w]],
                                  csem[b]).wait()

        for k in range(NCHUNK):
            pltpu.sync_copy(src_hbm.at[wid * NCHUNK + k], src_v)
            pltpu.sync_copy(dst_hbm.at[wid * NCHUNK + k], dst_v)

            fire_gather(0, 0)
            fire_gather(1, 1)
            wait_gather(0, 0)
            start_scatter(0, 0)
            fire_gather(2, 2)
            wait_gather(1, 1)
            start_scatter(1, 1)
            wait_scatter(0, 0)
            fire_gather(3, 0)

            @pl.loop(0, (CWIN - 4) // 3)
            def _(q):
                for db in range(3):
                    w = 2 + q * 3 + db
                    b = (2 + db) % 3
                    wait_gather(w, b)
                    start_scatter(w, b)
                    wait_scatter(w - 1, (b + 2) % 3)
                    fire_gather(w + 2, (b + 2) % 3)

            wait_gather(CWIN - 2, (CWIN - 2) % 3)
            start_scatter(CWIN - 2, (CWIN - 2) % 3)
            wait_scatter(CWIN - 3, (CWIN - 3) % 3)
            wait_gather(CWIN - 1, (CWIN - 1) % 3)
            start_scatter(CWIN - 1, (CWIN - 1) % 3)
            wait_scatter(CWIN - 2, (CWIN - 2) % 3)
            wait_scatter(CWIN - 1, (CWIN - 1) % 3)

        plsc.subcore_barrier()
        pltpu.sync_copy(
            acc_sh.at[pl.ds(stripe, ROWS_PER_SUB)],
            out_hbm.at[cid, pl.ds(stripe, ROWS_PER_SUB)])
        pltpu.sync_copy(
            cnt_sh.at[pl.ds(stripe, ROWS_PER_SUB)],
            cnt_hbm.at[cid, pl.ds(stripe, ROWS_PER_SUB)])

    return agg(x, src3, dst3, zf, zc)


def _tc_finish(acc, cnt, x, W_l, b_l, W_r):
    R = 1000

    def body(acc_ref, cnt_ref, x_ref, wl_ref, bl_ref, wr_ref, o_ref):
        summed = acc_ref[0] + acc_ref[1]
        counts = (cnt_ref[0] + cnt_ref[1])[:, :1]
        mean = summed / jnp.maximum(counts, 1.0)
        z = jnp.dot(mean, wl_ref[...], preferred_element_type=jnp.float32)
        z = z + bl_ref[...] + jnp.dot(x_ref[...], wr_ref[...],
                                      preferred_element_type=jnp.float32)
        o_ref[...] = jnp.maximum(z, 0.0)

    return pl.pallas_call(
        body,
        grid=(N_NODES // R,),
        in_specs=[
            pl.BlockSpec((NC, R, CH), lambda i: (0, i, 0)),
            pl.BlockSpec((NC, R, CNTW), lambda i: (0, i, 0)),
            pl.BlockSpec((R, CH), lambda i: (i, 0)),
            pl.BlockSpec((CH, CH), lambda i: (0, 0)),
            pl.BlockSpec((1, CH), lambda i: (0, 0)),
            pl.BlockSpec((CH, CH), lambda i: (0, 0)),
        ],
        out_specs=pl.BlockSpec((R, CH), lambda i: (i, 0)),
        out_shape=jax.ShapeDtypeStruct((N_NODES, CH), jnp.float32),
    )(acc, cnt, x, W_l, b_l.reshape(1, CH), W_r)


def kernel(x, edge_index, W_l, b_l, W_r):
    src3 = edge_index[0].reshape(NW * NCHUNK, CWIN, WIN)
    dst3 = edge_index[1].reshape(NW * NCHUNK, CWIN, WIN)
    zf = jnp.zeros((ROWS_PER_SUB, CH), jnp.float32)
    zc = jnp.zeros((ROWS_PER_SUB, CNTW), jnp.float32)
    acc, cnt = _sc_aggregate(x, src3, dst3, zf, zc)
    return _tc_finish(acc, cnt, x, W_l, b_l, W_r)
